# baseline (speedup 1.0000x reference)
import functools

import jax
import jax.numpy as jnp
from jax.experimental import pallas as pl
from jax.experimental.pallas import tpu as pltpu
from jax.experimental.pallas import tpu_sc as plsc

B = 16384
UD = 320
SD = 64
NC, NS = 2, 16
NW = NC * NS
B_PER_W = B // NW
CHUNK = 64
N_CHUNKS = B_PER_W // CHUNK
TCHUNK = 128
N_TCHUNKS = B_PER_W // TCHUNK
N_SIDE = 4
_MESH = plsc.VectorSubcoreMesh(core_axis_name="c", subcore_axis_name="s")


def _sc_main(uidx2, sidx2, user_table, side_table):
    @functools.partial(
        pl.kernel,
        out_type=(
            jax.ShapeDtypeStruct((B, 256), jnp.float32),
            jax.ShapeDtypeStruct((B * N_SIDE, 128), jnp.float32),
        ),
        mesh=_MESH,
        scratch_types=[
            pltpu.VMEM((N_CHUNKS, CHUNK), jnp.int32),
            pltpu.VMEM((N_CHUNKS * N_SIDE, CHUNK), jnp.int32),
            pltpu.VMEM((CHUNK, 256), jnp.float32),
            pltpu.VMEM((CHUNK, 256), jnp.float32),
            pltpu.VMEM((CHUNK * N_SIDE, 128), jnp.float32),
            pltpu.VMEM((CHUNK * N_SIDE, 128), jnp.float32),
            pltpu.SemaphoreType.DMA,
            pltpu.SemaphoreType.DMA,
        ],
    )
    def k(uidx_hbm, sidx_hbm, user_hbm, side_hbm, out_u, out_s,
          uidx_v, sidx_v, u0_v, u1_v, s0_v, s1_v, gsem, osem):
        wid = jax.lax.axis_index("s") * NC + jax.lax.axis_index("c")
        base = wid * B_PER_W
        pltpu.sync_copy(uidx_hbm.at[pl.ds(wid * N_CHUNKS, N_CHUNKS), :],
                        uidx_v)
        pltpu.sync_copy(
            sidx_hbm.at[pl.ds(wid * N_CHUNKS * N_SIDE, N_CHUNKS * N_SIDE), :],
            sidx_v)
        ubufs = [u0_v, u1_v]
        sbufs = [s0_v, s1_v]
        gets = [None, None]
        puts = [None, None]
        for c in range(N_CHUNKS):
            cur = c % 2
            if puts[cur] is not None:
                for cp in puts[cur]:
                    cp.wait()
            gets[cur] = [
                pltpu.async_copy(user_hbm.at[uidx_v.at[c], pl.ds(0, 256)],
                                 ubufs[cur], gsem)
            ] + [
                pltpu.async_copy(side_hbm.at[sidx_v.at[c * N_SIDE + j]],
                                 sbufs[cur].at[pl.ds(j * CHUNK, CHUNK), :],
                                 gsem)
                for j in range(N_SIDE)
            ]
            if c == 0:
                continue
            prev = (c - 1) % 2
            for cp in gets[prev]:
                cp.wait()
            row0 = base + (c - 1) * CHUNK
            puts[prev] = [
                pltpu.async_copy(ubufs[prev],
                                 out_u.at[pl.ds(row0, CHUNK), :], osem),
                pltpu.async_copy(sbufs[prev],
                                 out_s.at[pl.ds(row0 * N_SIDE,
                                                CHUNK * N_SIDE), :], osem),
            ]
        last = (N_CHUNKS - 1) % 2
        for cp in gets[last]:
            cp.wait()
        row0 = base + (N_CHUNKS - 1) * CHUNK
        pltpu.sync_copy(ubufs[last], out_u.at[pl.ds(row0, CHUNK), :])
        pltpu.sync_copy(sbufs[last],
                        out_s.at[pl.ds(row0 * N_SIDE, CHUNK * N_SIDE), :])
        if puts[1 - last] is not None:
            for cp in puts[1 - last]:
                cp.wait()

    return k(uidx2, sidx2, user_table, side_table)


def _sc_tail(uidx2t, tail_table):
    @functools.partial(
        pl.kernel,
        out_type=jax.ShapeDtypeStruct((B, 128), jnp.float32),
        mesh=_MESH,
        scratch_types=[
            pltpu.VMEM((N_TCHUNKS, TCHUNK), jnp.int32),
            pltpu.VMEM((TCHUNK, 128), jnp.float32),
            pltpu.VMEM((TCHUNK, 128), jnp.float32),
            pltpu.SemaphoreType.DMA,
            pltpu.SemaphoreType.DMA,
        ],
    )
    def k(uidx_hbm, tail_hbm, out_t, uidx_v, t0_v, t1_v, gsem, osem):
        wid = jax.lax.axis_index("s") * NC + jax.lax.axis_index("c")
        base = wid * B_PER_W
        pltpu.sync_copy(uidx_hbm.at[pl.ds(wid * N_TCHUNKS, N_TCHUNKS), :],
                        uidx_v)
        bufs = [t0_v, t1_v]
        gets = [None, None]
        puts = [None, None]
        for c in range(N_TCHUNKS):
            cur = c % 2
            if puts[cur] is not None:
                puts[cur].wait()
            gets[cur] = pltpu.async_copy(tail_hbm.at[uidx_v.at[c]], bufs[cur],
                                         gsem)
            if c == 0:
                continue
            prev = (c - 1) % 2
            gets[prev].wait()
            puts[prev] = pltpu.async_copy(
                bufs[prev],
                out_t.at[pl.ds(base + (c - 1) * TCHUNK, TCHUNK), :], osem)
        last = (N_TCHUNKS - 1) % 2
        gets[last].wait()
        pltpu.sync_copy(bufs[last],
                        out_t.at[pl.ds(base + (N_TCHUNKS - 1) * TCHUNK,
                                       TCHUNK), :])
        if puts[1 - last] is not None:
            puts[1 - last].wait()

    return k(uidx2t, tail_table)


def _tc_tail_table(user_table):
    n = user_table.shape[0]
    blk = 8192

    def body(in_ref, out_ref):
        val = in_ref[:, :SD]
        out_ref[:, :SD] = val
        out_ref[:, SD:] = jnp.zeros_like(val)

    return pl.pallas_call(
        body,
        grid=(pl.cdiv(n, blk),),
        in_specs=[pl.BlockSpec((blk, 128), lambda i: (i, 2))],
        out_specs=pl.BlockSpec((blk, 128), lambda i: (i, 0)),
        out_shape=jax.ShapeDtypeStruct((n, 128), jnp.float32),
    )(user_table)


def kernel(data, user_table, gender_table, age_table, occup_table, zip_table):
    idx = data[:, 0, :].astype(jnp.int32)
    uidx2 = idx[:, 0].reshape(B // CHUNK, CHUNK)
    uidx2t = idx[:, 0].reshape(B // TCHUNK, TCHUNK)

    side_table = jnp.concatenate(
        [gender_table, age_table, occup_table, zip_table], axis=0)
    side_table = jnp.pad(side_table, ((0, 0), (0, 128 - SD)))
    offs = jnp.array([0, 2, 2 + 7, 2 + 7 + 21], jnp.int32)
    sidx2 = (idx[:, 1:5] + offs).reshape(B * N_SIDE // CHUNK, CHUNK)

    u, s = _sc_main(uidx2, sidx2, user_table, side_table)
    tail_table = _tc_tail_table(user_table)
    t = _sc_tail(uidx2t, tail_table)
    side = s[:, :SD].reshape(B, N_SIDE * SD)
    return jnp.concatenate([u, t[:, :SD], side], axis=1)


# speedup vs baseline: 1.1347x; 1.1347x over previous
import functools

import jax
import jax.numpy as jnp
from jax.experimental import pallas as pl
from jax.experimental.pallas import tpu as pltpu
from jax.experimental.pallas import tpu_sc as plsc

B = 16384
UD = 320
SD = 64
NC, NS = 2, 16
NW = NC * NS
B_PER_W = B // NW
CHUNK = 64
N_CHUNKS = B_PER_W // CHUNK
TCHUNK = 128
N_TCHUNKS = B_PER_W // TCHUNK
N_SIDE = 4
_MESH = plsc.VectorSubcoreMesh(core_axis_name="c", subcore_axis_name="s")


def _sc_main(uidx2, gaidx2, ozidx2, user_table, pair_table):
    @functools.partial(
        pl.kernel,
        out_type=(
            jax.ShapeDtypeStruct((B, 256), jnp.float32),
            jax.ShapeDtypeStruct((B, 128), jnp.float32),
            jax.ShapeDtypeStruct((B, 128), jnp.float32),
        ),
        mesh=_MESH,
        scratch_types=[
            pltpu.VMEM((N_CHUNKS, CHUNK), jnp.int32),
            pltpu.VMEM((N_CHUNKS, CHUNK), jnp.int32),
            pltpu.VMEM((N_CHUNKS, CHUNK), jnp.int32),
            pltpu.VMEM((CHUNK, 256), jnp.float32),
            pltpu.VMEM((CHUNK, 256), jnp.float32),
            pltpu.VMEM((CHUNK, 128), jnp.float32),
            pltpu.VMEM((CHUNK, 128), jnp.float32),
            pltpu.VMEM((CHUNK, 128), jnp.float32),
            pltpu.VMEM((CHUNK, 128), jnp.float32),
            pltpu.SemaphoreType.DMA,
            pltpu.SemaphoreType.DMA,
        ],
    )
    def k(uidx_hbm, gaidx_hbm, ozidx_hbm, user_hbm, pair_hbm,
          out_u, out_ga, out_oz,
          uidx_v, gaidx_v, ozidx_v, u0_v, u1_v, ga0_v, ga1_v, oz0_v, oz1_v,
          gsem, osem):
        wid = jax.lax.axis_index("s") * NC + jax.lax.axis_index("c")
        base = wid * B_PER_W
        rows_w = pl.ds(wid * N_CHUNKS, N_CHUNKS)
        pltpu.sync_copy(uidx_hbm.at[rows_w, :], uidx_v)
        pltpu.sync_copy(gaidx_hbm.at[rows_w, :], gaidx_v)
        pltpu.sync_copy(ozidx_hbm.at[rows_w, :], ozidx_v)
        ubufs = [u0_v, u1_v]
        gabufs = [ga0_v, ga1_v]
        ozbufs = [oz0_v, oz1_v]
        gets = [None, None]
        puts = [None, None]
        for c in range(N_CHUNKS):
            cur = c % 2
            if puts[cur] is not None:
                for cp in puts[cur]:
                    cp.wait()
            gets[cur] = [
                pltpu.async_copy(user_hbm.at[uidx_v.at[c], pl.ds(0, 256)],
                                 ubufs[cur], gsem),
                pltpu.async_copy(pair_hbm.at[gaidx_v.at[c]], gabufs[cur],
                                 gsem),
                pltpu.async_copy(pair_hbm.at[ozidx_v.at[c]], ozbufs[cur],
                                 gsem),
            ]
            if c == 0:
                continue
            prev = (c - 1) % 2
            for cp in gets[prev]:
                cp.wait()
            rows = pl.ds(base + (c - 1) * CHUNK, CHUNK)
            puts[prev] = [
                pltpu.async_copy(ubufs[prev], out_u.at[rows, :], osem),
                pltpu.async_copy(gabufs[prev], out_ga.at[rows, :], osem),
                pltpu.async_copy(ozbufs[prev], out_oz.at[rows, :], osem),
            ]
        last = (N_CHUNKS - 1) % 2
        for cp in gets[last]:
            cp.wait()
        rows = pl.ds(base + (N_CHUNKS - 1) * CHUNK, CHUNK)
        pltpu.sync_copy(ubufs[last], out_u.at[rows, :])
        pltpu.sync_copy(gabufs[last], out_ga.at[rows, :])
        pltpu.sync_copy(ozbufs[last], out_oz.at[rows, :])
        if puts[1 - last] is not None:
            for cp in puts[1 - last]:
                cp.wait()

    return k(uidx2, gaidx2, ozidx2, user_table, pair_table)


def _sc_tail(uidx2t, tail_table):
    @functools.partial(
        pl.kernel,
        out_type=jax.ShapeDtypeStruct((B, 128), jnp.float32),
        mesh=_MESH,
        scratch_types=[
            pltpu.VMEM((N_TCHUNKS, TCHUNK), jnp.int32),
            pltpu.VMEM((TCHUNK, 128), jnp.float32),
            pltpu.VMEM((TCHUNK, 128), jnp.float32),
            pltpu.SemaphoreType.DMA,
            pltpu.SemaphoreType.DMA,
        ],
    )
    def k(uidx_hbm, tail_hbm, out_t, uidx_v, t0_v, t1_v, gsem, osem):
        wid = jax.lax.axis_index("s") * NC + jax.lax.axis_index("c")
        base = wid * B_PER_W
        pltpu.sync_copy(uidx_hbm.at[pl.ds(wid * N_TCHUNKS, N_TCHUNKS), :],
                        uidx_v)
        bufs = [t0_v, t1_v]
        gets = [None, None]
        puts = [None, None]
        for c in range(N_TCHUNKS):
            cur = c % 2
            if puts[cur] is not None:
                puts[cur].wait()
            gets[cur] = pltpu.async_copy(tail_hbm.at[uidx_v.at[c]], bufs[cur],
                                         gsem)
            if c == 0:
                continue
            prev = (c - 1) % 2
            gets[prev].wait()
            puts[prev] = pltpu.async_copy(
                bufs[prev],
                out_t.at[pl.ds(base + (c - 1) * TCHUNK, TCHUNK), :], osem)
        last = (N_TCHUNKS - 1) % 2
        gets[last].wait()
        pltpu.sync_copy(bufs[last],
                        out_t.at[pl.ds(base + (N_TCHUNKS - 1) * TCHUNK,
                                       TCHUNK), :])
        if puts[1 - last] is not None:
            puts[1 - last].wait()

    return k(uidx2t, tail_table)


def _tc_tail_table(user_table):
    n = user_table.shape[0]
    blk = 8192

    def body(in_ref, out_ref):
        val = in_ref[:, :SD]
        out_ref[:, :SD] = val
        out_ref[:, SD:] = jnp.zeros_like(val)

    return pl.pallas_call(
        body,
        grid=(pl.cdiv(n, blk),),
        in_specs=[pl.BlockSpec((blk, 128), lambda i: (i, 2))],
        out_specs=pl.BlockSpec((blk, 128), lambda i: (i, 0)),
        out_shape=jax.ShapeDtypeStruct((n, 128), jnp.float32),
    )(user_table)


def kernel(data, user_table, gender_table, age_table, occup_table, zip_table):
    idx = data[:, 0, :].astype(jnp.int32)
    uidx2 = idx[:, 0].reshape(B // CHUNK, CHUNK)
    uidx2t = idx[:, 0].reshape(B // TCHUNK, TCHUNK)

    n_ga = 2 * 7
    n_oz = 21 * 3439
    ga_tbl = jnp.concatenate(
        [jnp.repeat(gender_table, 7, axis=0), jnp.tile(age_table, (2, 1))],
        axis=1)
    oz_tbl = jnp.concatenate(
        [jnp.repeat(occup_table, 3439, axis=0),
         jnp.tile(zip_table, (21, 1))], axis=1)
    pair_table = jnp.concatenate([ga_tbl, oz_tbl], axis=0)
    gaidx2 = (idx[:, 1] * 7 + idx[:, 2]).reshape(B // CHUNK, CHUNK)
    ozidx2 = (n_ga + idx[:, 3] * 3439 + idx[:, 4]).reshape(B // CHUNK, CHUNK)

    u, ga, oz = _sc_main(uidx2, gaidx2, ozidx2, user_table, pair_table)
    tail_table = _tc_tail_table(user_table)
    t = _sc_tail(uidx2t, tail_table)
    return jnp.concatenate([u, t[:, :SD], ga, oz], axis=1)


# speedup vs baseline: 1.1502x; 1.0137x over previous
import functools

import jax
import jax.numpy as jnp
from jax.experimental import pallas as pl
from jax.experimental.pallas import tpu as pltpu
from jax.experimental.pallas import tpu_sc as plsc

B = 16384
UD = 320
SD = 64
NC, NS = 2, 16
NW = NC * NS
B_PER_W = B // NW
CHUNK = 64
N_CHUNKS = B_PER_W // CHUNK
TCHUNK = 128
N_TCHUNKS = B_PER_W // TCHUNK
N_SIDE = 4
_MESH = plsc.VectorSubcoreMesh(core_axis_name="c", subcore_axis_name="s")


def _sc_main(uidx2, gaidx2, ozidx2, user_table, pair_table):
    @functools.partial(
        pl.kernel,
        out_type=(
            jax.ShapeDtypeStruct((B, 256), jnp.float32),
            jax.ShapeDtypeStruct((B, 128), jnp.float32),
            jax.ShapeDtypeStruct((B, 128), jnp.float32),
        ),
        mesh=_MESH,
        scratch_types=[
            pltpu.VMEM((N_CHUNKS, CHUNK), jnp.int32),
            pltpu.VMEM((N_CHUNKS, CHUNK), jnp.int32),
            pltpu.VMEM((N_CHUNKS, CHUNK), jnp.int32),
            pltpu.VMEM((CHUNK, 256), jnp.float32),
            pltpu.VMEM((CHUNK, 256), jnp.float32),
            pltpu.VMEM((CHUNK, 128), jnp.float32),
            pltpu.VMEM((CHUNK, 128), jnp.float32),
            pltpu.VMEM((CHUNK, 128), jnp.float32),
            pltpu.VMEM((CHUNK, 128), jnp.float32),
            pltpu.SemaphoreType.DMA,
            pltpu.SemaphoreType.DMA,
        ],
    )
    def k(uidx_hbm, gaidx_hbm, ozidx_hbm, user_hbm, pair_hbm,
          out_u, out_ga, out_oz,
          uidx_v, gaidx_v, ozidx_v, u0_v, u1_v, ga0_v, ga1_v, oz0_v, oz1_v,
          gsem, osem):
        wid = jax.lax.axis_index("s") * NC + jax.lax.axis_index("c")
        base = wid * B_PER_W
        rows_w = pl.ds(wid * N_CHUNKS, N_CHUNKS)
        pltpu.sync_copy(uidx_hbm.at[rows_w, :], uidx_v)
        pltpu.sync_copy(gaidx_hbm.at[rows_w, :], gaidx_v)
        pltpu.sync_copy(ozidx_hbm.at[rows_w, :], ozidx_v)
        ubufs = [u0_v, u1_v]
        gabufs = [ga0_v, ga1_v]
        ozbufs = [oz0_v, oz1_v]
        gets = [None, None]
        puts = [None, None]
        for c in range(N_CHUNKS):
            cur = c % 2
            if puts[cur] is not None:
                for cp in puts[cur]:
                    cp.wait()
            gets[cur] = [
                pltpu.async_copy(user_hbm.at[uidx_v.at[c], pl.ds(0, 256)],
                                 ubufs[cur], gsem),
                pltpu.async_copy(pair_hbm.at[gaidx_v.at[c]], gabufs[cur],
                                 gsem),
                pltpu.async_copy(pair_hbm.at[ozidx_v.at[c]], ozbufs[cur],
                                 gsem),
            ]
            if c == 0:
                continue
            prev = (c - 1) % 2
            for cp in gets[prev]:
                cp.wait()
            rows = pl.ds(base + (c - 1) * CHUNK, CHUNK)
            puts[prev] = [
                pltpu.async_copy(ubufs[prev], out_u.at[rows, :], osem),
                pltpu.async_copy(gabufs[prev], out_ga.at[rows, :], osem),
                pltpu.async_copy(ozbufs[prev], out_oz.at[rows, :], osem),
            ]
        last = (N_CHUNKS - 1) % 2
        for cp in gets[last]:
            cp.wait()
        rows = pl.ds(base + (N_CHUNKS - 1) * CHUNK, CHUNK)
        pltpu.sync_copy(ubufs[last], out_u.at[rows, :])
        pltpu.sync_copy(gabufs[last], out_ga.at[rows, :])
        pltpu.sync_copy(ozbufs[last], out_oz.at[rows, :])
        if puts[1 - last] is not None:
            for cp in puts[1 - last]:
                cp.wait()

    return k(uidx2, gaidx2, ozidx2, user_table, pair_table)


def _sc_tail(uidx2t, tail_table):
    @functools.partial(
        pl.kernel,
        out_type=jax.ShapeDtypeStruct((B, 128), jnp.float32),
        mesh=_MESH,
        scratch_types=[
            pltpu.VMEM((N_TCHUNKS, TCHUNK), jnp.int32),
            pltpu.VMEM((TCHUNK, 128), jnp.float32),
            pltpu.VMEM((TCHUNK, 128), jnp.float32),
            pltpu.SemaphoreType.DMA,
            pltpu.SemaphoreType.DMA,
        ],
    )
    def k(uidx_hbm, tail_hbm, out_t, uidx_v, t0_v, t1_v, gsem, osem):
        wid = jax.lax.axis_index("s") * NC + jax.lax.axis_index("c")
        base = wid * B_PER_W
        pltpu.sync_copy(uidx_hbm.at[pl.ds(wid * N_TCHUNKS, N_TCHUNKS), :],
                        uidx_v)
        bufs = [t0_v, t1_v]
        gets = [None, None]
        puts = [None, None]
        for c in range(N_TCHUNKS):
            cur = c % 2
            if puts[cur] is not None:
                puts[cur].wait()
            gets[cur] = pltpu.async_copy(tail_hbm.at[uidx_v.at[c]], bufs[cur],
                                         gsem)
            if c == 0:
                continue
            prev = (c - 1) % 2
            gets[prev].wait()
            puts[prev] = pltpu.async_copy(
                bufs[prev],
                out_t.at[pl.ds(base + (c - 1) * TCHUNK, TCHUNK), :], osem)
        last = (N_TCHUNKS - 1) % 2
        gets[last].wait()
        pltpu.sync_copy(bufs[last],
                        out_t.at[pl.ds(base + (N_TCHUNKS - 1) * TCHUNK,
                                       TCHUNK), :])
        if puts[1 - last] is not None:
            puts[1 - last].wait()

    return k(uidx2t, tail_table)


def _tc_tail_table(user_table):
    n = user_table.shape[0]
    blk = 8192

    def body(in_ref, out_ref):
        val = in_ref[:, :SD]
        out_ref[:, :SD] = val
        out_ref[:, SD:] = jnp.zeros_like(val)

    return pl.pallas_call(
        body,
        grid=(pl.cdiv(n, blk),),
        in_specs=[pl.BlockSpec((blk, 128), lambda i: (i, 2))],
        out_specs=pl.BlockSpec((blk, 128), lambda i: (i, 0)),
        out_shape=jax.ShapeDtypeStruct((n, 128), jnp.float32),
    )(user_table)


def _tc_assemble_t(u, t, ga, oz):
    blk = 512

    def body(u_ref, t_ref, ga_ref, oz_ref, out_ref):
        out_ref[pl.ds(0, 256), :] = jnp.transpose(u_ref[...])
        out_ref[pl.ds(256, SD), :] = jnp.transpose(t_ref[:, :SD])
        out_ref[pl.ds(256 + SD, 128), :] = jnp.transpose(ga_ref[...])
        out_ref[pl.ds(256 + SD + 128, 128), :] = jnp.transpose(oz_ref[...])

    return pl.pallas_call(
        body,
        grid=(B // blk,),
        in_specs=[
            pl.BlockSpec((blk, 256), lambda i: (i, 0)),
            pl.BlockSpec((blk, 128), lambda i: (i, 0)),
            pl.BlockSpec((blk, 128), lambda i: (i, 0)),
            pl.BlockSpec((blk, 128), lambda i: (i, 0)),
        ],
        out_specs=pl.BlockSpec((576, blk), lambda i: (0, i)),
        out_shape=jax.ShapeDtypeStruct((576, B), jnp.float32),
    )(u, t, ga, oz)


def kernel(data, user_table, gender_table, age_table, occup_table, zip_table):
    idx = data[:, 0, :].astype(jnp.int32)
    uidx2 = idx[:, 0].reshape(B // CHUNK, CHUNK)
    uidx2t = idx[:, 0].reshape(B // TCHUNK, TCHUNK)

    n_ga = 2 * 7
    n_oz = 21 * 3439
    ga_tbl = jnp.concatenate(
        [jnp.repeat(gender_table, 7, axis=0), jnp.tile(age_table, (2, 1))],
        axis=1)
    oz_tbl = jnp.concatenate(
        [jnp.repeat(occup_table, 3439, axis=0),
         jnp.tile(zip_table, (21, 1))], axis=1)
    pair_table = jnp.concatenate([ga_tbl, oz_tbl], axis=0)
    gaidx2 = (idx[:, 1] * 7 + idx[:, 2]).reshape(B // CHUNK, CHUNK)
    ozidx2 = (n_ga + idx[:, 3] * 3439 + idx[:, 4]).reshape(B // CHUNK, CHUNK)

    u, ga, oz = _sc_main(uidx2, gaidx2, ozidx2, user_table, pair_table)
    tail_table = _tc_tail_table(user_table)
    t = _sc_tail(uidx2t, tail_table)
    return _tc_assemble_t(u, t, ga, oz).T
